# Initial kernel scaffold; baseline (speedup 1.0000x reference)
import jax
import jax.numpy as jnp
from jax.experimental import pallas as pl


def kernel(input_ids, word_embeddings):
    raise NotImplementedError("write your pallas kernel here")


# speedup vs baseline: 1.7622x; 1.7622x over previous
import functools

import jax
import jax.numpy as jnp
from jax import lax
from jax.experimental import pallas as pl
from jax.experimental.pallas import tpu as pltpu
from jax.experimental.pallas import tpu_sc as plsc

VOCAB = 100000
D = 2048
BATCH = 4
SEQ = 4096
B_TOT = BATCH * SEQ

NC = 2
NS = 16
NW = NC * NS
BPW = B_TOT // NW

C = 16
NCHUNK = BPW // C
NBUF = 2


def _emb_body(idx_hbm, table_hbm, out_hbm, idx_v, rows_v, gsem):
    wid = lax.axis_index("s") * NC + lax.axis_index("c")
    base = wid * BPW

    pltpu.sync_copy(idx_hbm.at[pl.ds(base, BPW)], idx_v)

    def _gather_start(c, b):
        pltpu.async_copy(
            table_hbm.at[idx_v.at[pl.ds(c * C, C)]], rows_v.at[b], gsem.at[b]
        )

    def _gather_wait(c, b):
        pltpu.make_async_copy(
            table_hbm.at[idx_v.at[pl.ds(c * C, C)]], rows_v.at[b], gsem.at[b]
        ).wait()

    for b in range(NBUF):
        _gather_start(b, b)

    @pl.loop(0, NCHUNK - NBUF, step=NBUF)
    def _steady(i):
        for b in range(NBUF):
            c = i + b
            _gather_wait(c, b)
            pltpu.sync_copy(rows_v.at[b], out_hbm.at[pl.ds(base + c * C, C)])
            _gather_start(c + NBUF, b)

    for b in range(NBUF):
        c = NCHUNK - NBUF + b
        _gather_wait(c, b)
        pltpu.sync_copy(rows_v.at[b], out_hbm.at[pl.ds(base + c * C, C)])


@jax.jit
def _emb(flat_ids, table):
    mesh = plsc.VectorSubcoreMesh(core_axis_name="c", subcore_axis_name="s")
    run = pl.kernel(
        _emb_body,
        out_type=jax.ShapeDtypeStruct((B_TOT, D), jnp.float32),
        mesh=mesh,
        scratch_types=[
            pltpu.VMEM((BPW,), jnp.int32),
            pltpu.VMEM((NBUF, C, D), jnp.float32),
            pltpu.SemaphoreType.DMA((NBUF,)),
        ],
    )
    return run(flat_ids, table)


def kernel(input_ids, word_embeddings):
    flat_ids = input_ids.reshape(-1).astype(jnp.int32)
    out = _emb(flat_ids, word_embeddings)
    return out.reshape(input_ids.shape[0], input_ids.shape[1], D)


# speedup vs baseline: 1.7704x; 1.0047x over previous
import functools

import jax
import jax.numpy as jnp
from jax import lax
from jax.experimental import pallas as pl
from jax.experimental.pallas import tpu as pltpu
from jax.experimental.pallas import tpu_sc as plsc

VOCAB = 100000
D = 2048
BATCH = 4
SEQ = 4096
B_TOT = BATCH * SEQ

NC = 2
NS = 16
NW = NC * NS
BPW = B_TOT // NW

C = 8
NCHUNK = BPW // C
NBUF = 4


def _emb_body(idx_hbm, table_hbm, out_hbm, idx_v, rows_v, gsem):
    wid = lax.axis_index("s") * NC + lax.axis_index("c")
    base = wid * BPW

    pltpu.sync_copy(idx_hbm.at[pl.ds(base, BPW)], idx_v)

    def _gather_start(c, b):
        pltpu.async_copy(
            table_hbm.at[idx_v.at[pl.ds(c * C, C)]], rows_v.at[b], gsem.at[b]
        )

    def _gather_wait(c, b):
        pltpu.make_async_copy(
            table_hbm.at[idx_v.at[pl.ds(c * C, C)]], rows_v.at[b], gsem.at[b]
        ).wait()

    for b in range(NBUF):
        _gather_start(b, b)

    @pl.loop(0, NCHUNK - NBUF, step=NBUF)
    def _steady(i):
        for b in range(NBUF):
            c = i + b
            _gather_wait(c, b)
            pltpu.sync_copy(rows_v.at[b], out_hbm.at[pl.ds(base + c * C, C)])
            _gather_start(c + NBUF, b)

    for b in range(NBUF):
        c = NCHUNK - NBUF + b
        _gather_wait(c, b)
        pltpu.sync_copy(rows_v.at[b], out_hbm.at[pl.ds(base + c * C, C)])


@jax.jit
def _emb(flat_ids, table):
    mesh = plsc.VectorSubcoreMesh(core_axis_name="c", subcore_axis_name="s")
    run = pl.kernel(
        _emb_body,
        out_type=jax.ShapeDtypeStruct((B_TOT, D), jnp.float32),
        mesh=mesh,
        scratch_types=[
            pltpu.VMEM((BPW,), jnp.int32),
            pltpu.VMEM((NBUF, C, D), jnp.float32),
            pltpu.SemaphoreType.DMA((NBUF,)),
        ],
    )
    return run(flat_ids, table)


def kernel(input_ids, word_embeddings):
    flat_ids = input_ids.reshape(-1).astype(jnp.int32)
    out = _emb(flat_ids, word_embeddings)
    return out.reshape(input_ids.shape[0], input_ids.shape[1], D)
